# baseline (speedup 1.0000x reference)
import functools

import jax
import jax.numpy as jnp
from jax import lax
from jax.experimental import pallas as pl
from jax.experimental.pallas import tpu as pltpu
from jax.experimental.pallas import tpu_sc as plsc

_LANES = 16
_UNROLL = 8


def _build(B, S, D):
    mesh = plsc.VectorSubcoreMesh(
        core_axis_name="c", subcore_axis_name="s",
        num_cores=1, num_subcores=B,
    )

    @functools.partial(
        pl.kernel,
        mesh=mesh,
        out_type=jax.ShapeDtypeStruct((B, D), jnp.float32),
        scratch_types=[
            pltpu.VMEM((S,), jnp.int32),
        ],
    )
    def body(emb_hbm, mask_hbm, out_hbm, mask_v):
        b = lax.axis_index("s")
        with jax.named_scope("mask_dma"):
            pltpu.sync_copy(mask_hbm.at[b], mask_v)

        span = _LANES * _UNROLL

        def step(i, accs):
            base = i * span
            return tuple(
                a + mask_v[pl.ds(base + k * _LANES, _LANES)]
                for k, a in enumerate(accs)
            )

        with jax.named_scope("reduce_loop"):
            accs = lax.fori_loop(
                0, S // span, step,
                tuple(jnp.zeros((_LANES,), jnp.int32) for _ in range(_UNROLL)),
            )
            acc = accs[0]
            for a in accs[1:]:
                acc = acc + a
        with jax.named_scope("lane_extract"):
            total = acc[0]
            for lane in range(1, _LANES):
                total = total + acc[lane]

        idx = b * S + total - 1
        with jax.named_scope("row_dma"):
            pltpu.sync_copy(emb_hbm.at[idx], out_hbm.at[b])

    return body


def kernel(token_embeddings, attention_mask):
    B, S, D = token_embeddings.shape
    emb2d = token_embeddings.reshape(B * S, D)
    return _build(B, S, D)(emb2d, attention_mask)


# speedup vs baseline: 1.0787x; 1.0787x over previous
import functools

import jax
import jax.numpy as jnp
from jax import lax
from jax.experimental import pallas as pl
from jax.experimental.pallas import tpu as pltpu
from jax.experimental.pallas import tpu_sc as plsc

_LANES = 16
_UNROLL = 8


def _build(B, S, D):
    mesh = plsc.VectorSubcoreMesh(
        core_axis_name="c", subcore_axis_name="s",
        num_cores=1, num_subcores=B,
    )

    @functools.partial(
        pl.kernel,
        mesh=mesh,
        out_type=jax.ShapeDtypeStruct((B, D), jnp.float32),
        scratch_types=[
            pltpu.VMEM((S,), jnp.int32),
            pltpu.VMEM((D,), jnp.float32),
        ],
    )
    def body(emb_hbm, mask_hbm, out_hbm, mask_v, row_v):
        b = lax.axis_index("s")
        with jax.named_scope("mask_dma"):
            pltpu.sync_copy(mask_hbm.at[b], mask_v)

        span = _LANES * _UNROLL

        def step(i, accs):
            base = i * span
            return tuple(
                a + mask_v[pl.ds(base + k * _LANES, _LANES)]
                for k, a in enumerate(accs)
            )

        with jax.named_scope("reduce_loop"):
            accs = lax.fori_loop(
                0, S // span, step,
                tuple(jnp.zeros((_LANES,), jnp.int32) for _ in range(_UNROLL)),
            )
            acc = accs[0]
            for a in accs[1:]:
                acc = acc + a
        with jax.named_scope("lane_extract"):
            total = acc[0]
            for lane in range(1, _LANES):
                total = total + acc[lane]

        idx = b * S + total - 1
        with jax.named_scope("row_dma_in"):
            pltpu.sync_copy(emb_hbm.at[idx], row_v)
        with jax.named_scope("row_dma_out"):
            pltpu.sync_copy(row_v, out_hbm.at[b])

    return body


def kernel(token_embeddings, attention_mask):
    B, S, D = token_embeddings.shape
    emb2d = token_embeddings.reshape(B * S, D)
    return _build(B, S, D)(emb2d, attention_mask)


# speedup vs baseline: 1.1047x; 1.0241x over previous
import functools

import jax
import jax.numpy as jnp
from jax import lax
from jax.experimental import pallas as pl
from jax.experimental.pallas import tpu as pltpu
from jax.experimental.pallas import tpu_sc as plsc

_LANES = 16
_UNROLL = 8
_WPB = 4
_MCH = 2


def _build(B, S, D):
    mesh = plsc.VectorSubcoreMesh(
        core_axis_name="c", subcore_axis_name="s",
        num_cores=1, num_subcores=B * _WPB,
    )
    dchunk = D // _WPB
    mchunk = S // _MCH
    span = _LANES * _UNROLL

    @functools.partial(
        pl.kernel,
        mesh=mesh,
        out_type=jax.ShapeDtypeStruct((B, D), jnp.float32),
        scratch_types=[
            pltpu.VMEM((S,), jnp.int32),
            pltpu.VMEM((dchunk,), jnp.float32),
        ]
        + [pltpu.SemaphoreType.DMA for _ in range(_MCH)],
    )
    def body(emb_hbm, mask_hbm, out_hbm, mask_v, row_v, *sems):
        sid = lax.axis_index("s")
        b = sid // _WPB
        q = sid % _WPB

        with jax.named_scope("mask_dma_start"):
            copies = [
                pltpu.async_copy(
                    mask_hbm.at[b, pl.ds(k * mchunk, mchunk)],
                    mask_v.at[pl.ds(k * mchunk, mchunk)],
                    sems[k],
                )
                for k in range(_MCH)
            ]

        def reduce_chunk(k, accs):
            def step(i, accs):
                base = k * mchunk + i * span
                return tuple(
                    a + mask_v[pl.ds(base + u * _LANES, _LANES)]
                    for u, a in enumerate(accs)
                )

            return lax.fori_loop(0, mchunk // span, step, accs)

        accs = tuple(jnp.zeros((_LANES,), jnp.int32) for _ in range(_UNROLL))
        for k in range(_MCH):
            with jax.named_scope("mask_wait"):
                copies[k].wait()
            with jax.named_scope("reduce_loop"):
                accs = reduce_chunk(k, accs)

        acc = accs[0]
        for a in accs[1:]:
            acc = acc + a
        total = acc[0]
        for lane in range(1, _LANES):
            total = total + acc[lane]

        idx = b * S + total - 1
        off = q * dchunk
        with jax.named_scope("row_dma_in"):
            pltpu.sync_copy(emb_hbm.at[idx, pl.ds(off, dchunk)], row_v)
        with jax.named_scope("row_dma_out"):
            pltpu.sync_copy(row_v, out_hbm.at[b, pl.ds(off, dchunk)])

    return body


def kernel(token_embeddings, attention_mask):
    B, S, D = token_embeddings.shape
    emb2d = token_embeddings.reshape(B * S, D)
    return _build(B, S, D)(emb2d, attention_mask)


# speedup vs baseline: 1.1149x; 1.0092x over previous
import functools

import jax
import jax.numpy as jnp
from jax import lax
from jax.experimental import pallas as pl
from jax.experimental.pallas import tpu as pltpu
from jax.experimental.pallas import tpu_sc as plsc

_LANES = 16
_UNROLL = 8
_WPB = 4
_MCH = 2


def _build(B, S, D):
    mesh = plsc.VectorSubcoreMesh(
        core_axis_name="c", subcore_axis_name="s",
        num_cores=1, num_subcores=B * _WPB,
    )
    dchunk = D // _WPB
    mchunk = S // _MCH
    span = _LANES * _UNROLL

    @functools.partial(
        pl.kernel,
        mesh=mesh,
        out_type=jax.ShapeDtypeStruct((B, D), jnp.float32),
        scratch_types=[
            pltpu.VMEM((S,), jnp.int32),
            pltpu.VMEM((dchunk,), jnp.float32),
        ]
        + [pltpu.SemaphoreType.DMA for _ in range(_MCH)],
    )
    def body(emb_hbm, mask_hbm, out_hbm, mask_v, row_v, *sems):
        sid = lax.axis_index("s")
        b = sid // _WPB
        q = sid % _WPB

        copies = [
            pltpu.async_copy(
                mask_hbm.at[b, pl.ds(k * mchunk, mchunk)],
                mask_v.at[pl.ds(k * mchunk, mchunk)],
                sems[k],
            )
            for k in range(_MCH)
        ]

        def reduce_chunk(k, accs):
            def step(i, accs):
                base = k * mchunk + i * span
                return tuple(
                    a + mask_v[pl.ds(base + u * _LANES, _LANES)]
                    for u, a in enumerate(accs)
                )

            return lax.fori_loop(0, mchunk // span, step, accs)

        accs = tuple(jnp.zeros((_LANES,), jnp.int32) for _ in range(_UNROLL))
        for k in range(_MCH):
            copies[k].wait()
            accs = reduce_chunk(k, accs)

        acc = accs[0]
        for a in accs[1:]:
            acc = acc + a
        total = acc[0]
        for lane in range(1, _LANES):
            total = total + acc[lane]

        idx = b * S + total - 1
        off = q * dchunk
        pltpu.sync_copy(emb_hbm.at[idx, pl.ds(off, dchunk)], row_v)
        pltpu.sync_copy(row_v, out_hbm.at[b, pl.ds(off, dchunk)])

    return body


def kernel(token_embeddings, attention_mask):
    B, S, D = token_embeddings.shape
    emb2d = token_embeddings.reshape(B * S, D)
    return _build(B, S, D)(emb2d, attention_mask)
